# baseline (speedup 1.0000x reference)
import functools

import jax
import jax.numpy as jnp
import numpy as np
from jax import lax
from jax.experimental import pallas as pl
from jax.experimental.pallas import tpu as pltpu
from jax.experimental.pallas import tpu_sc as plsc

N = 10000
E = 160000
D = 64
T = 64
NH = 4
HD = 16
FF = 256
M = 100
NL = 6

F32 = jnp.float32
BF16 = jnp.bfloat16

_NC = 2
_NS = 16
_NW = _NC * _NS
_CB = 128
_CR = E // _CB
_BASE = _CR // _NW
_EXTRA = _CR - _BASE * _NW
_NPT = N // _NS

def _sc_mesh():
    return plsc.VectorSubcoreMesh(core_axis_name="c", subcore_axis_name="s",
                                  num_cores=_NC, num_subcores=_NS)


def _ln_in(xb, g, b):
    m = jnp.mean(xb, axis=-1, keepdims=True)
    v = jnp.mean((xb - m) ** 2, axis=-1, keepdims=True)
    return (xb - m) / jnp.sqrt(v + 1e-5) * g + b


def _sc_gather2(hb, si2, di2):
    @functools.partial(
        pl.kernel,
        out_type=[jax.ShapeDtypeStruct((E, 128), F32),
                  jax.ShapeDtypeStruct((E, 128), F32)],
        mesh=_sc_mesh(),
        scratch_types=[
            pltpu.VMEM((_CB,), jnp.int32),
            pltpu.VMEM((_CB,), jnp.int32),
            pltpu.VMEM((_CB, 128), F32),
            pltpu.VMEM((_CB, 128), F32),
            pltpu.SemaphoreType.DMA,
            pltpu.SemaphoreType.DMA,
        ],
    )
    def gk(h_hbm, si_hbm, di_hbm, os_hbm, od_hbm, sib, dib, sbuf, dbuf, ssem, dsem):
        cid = lax.axis_index("c")
        sid = lax.axis_index("s")
        wid = sid * _NC + cid
        nk = jnp.where(wid < _EXTRA, _BASE + 1, _BASE)

        def body(k, carry):
            r = wid + _NW * k
            pltpu.sync_copy(si_hbm.at[r], sib)
            pltpu.sync_copy(di_hbm.at[r], dib)
            cs = pltpu.async_copy(h_hbm.at[sib], sbuf, ssem)
            cd = pltpu.async_copy(h_hbm.at[dib], dbuf, dsem)
            cs.wait()
            pltpu.sync_copy(sbuf, os_hbm.at[pl.ds(r * _CB, _CB)])
            cd.wait()
            pltpu.sync_copy(dbuf, od_hbm.at[pl.ds(r * _CB, _CB)])
            return carry

        lax.fori_loop(0, nk, body, 0)

    return gk(hb, si2, di2)


def _sc_scatter_add(we, di2):
    @functools.partial(
        pl.kernel,
        out_type=jax.ShapeDtypeStruct((_NC, N, 128), F32),
        mesh=_sc_mesh(),
        scratch_types=[
            pltpu.VMEM((_CB,), jnp.int32),
            pltpu.VMEM((_CB, 128), F32),
            pltpu.VMEM_SHARED((N, 128), F32),
        ],
    )
    def sk(we_hbm, di_hbm, out_hbm, dib, wbuf, shared):
        cid = lax.axis_index("c")
        sid = lax.axis_index("s")
        wid = sid * _NC + cid
        nk = jnp.where(wid < _EXTRA, _BASE + 1, _BASE)

        def zb(i, carry):
            r = i // 8
            c2 = (i % 8) * 16
            wbuf[r, pl.ds(c2, 16)] = jnp.zeros((16,), F32)
            return carry

        lax.fori_loop(0, _CB * 8, zb, 0)
        start = jnp.minimum(624 * sid, N - 640)
        for j in range(5):
            pltpu.sync_copy(wbuf, shared.at[pl.ds(start + j * _CB, _CB)])
        plsc.subcore_barrier()

        def body(k, carry):
            r = wid + _NW * k
            pltpu.sync_copy(we_hbm.at[pl.ds(r * _CB, _CB)], wbuf)
            pltpu.sync_copy(di_hbm.at[r], dib)
            pltpu.sync_copy(wbuf, shared.at[dib], add=True)
            return carry

        lax.fori_loop(0, nk, body, 0)
        plsc.subcore_barrier()
        for j in range(5):
            pltpu.sync_copy(shared.at[pl.ds(start + j * _CB, _CB)],
                            out_hbm.at[cid].at[pl.ds(start + j * _CB, _CB)])

    return sk(we, di2)


_BN = 2000
_BE = 4000


def _full(shape):
    return pl.BlockSpec(shape, lambda i: tuple(0 for _ in shape))


def _tc_node_encode(xin, t2, wt):
    names = ["tw1", "tb1", "tw2", "tb2",
             "vw1", "vb1", "vw2", "vb2", "vw3", "vb3",
             "nlg", "nlb", "nag", "nab", "naw", "nabias",
             "eaw", "eabias"]

    def body(xb, t_r, tw1, tb1, tw2, tb2, vw1, vb1, vw2, vb2, vw3, vb3,
             nlg, nlb, nag, nab, naw, nabias, eaw, eabias, h_o, hb_o, sse_o):
        tau = jax.nn.silu(t_r[...] @ tw1[...] + tb1[...]) @ tw2[...] + tb2[...]
        ss_n = tau @ naw[...] + nabias[...]
        ss_e = tau @ eaw[...] + eabias[...]
        sse_o[...] = ss_e
        h = xb[...]
        h = jax.nn.relu(h @ vw1[...] + vb1[...])
        h = jax.nn.relu(h @ vw2[...] + vb2[...])
        h = jax.nn.relu(h @ vw3[...] + vb3[...])
        h = _ln_in(h, nlg[...], nlb[...])
        scale = ss_n[:, :D]
        shift = ss_n[:, D:]
        h = _ln_in(h, nag[...], nab[...]) * (1.0 + scale) + shift
        h_o[...] = h
        hb_o[...] = jnp.concatenate(
            [h, jnp.zeros((h.shape[0], 128 - D), F32)], axis=-1)

    specs = [pl.BlockSpec((_BN, 4), lambda i: (i, 0)), _full((1, 1))]
    specs += [_full(wt[n].shape) for n in names]
    return pl.pallas_call(
        body,
        grid=(N // _BN,),
        in_specs=specs,
        out_specs=[pl.BlockSpec((_BN, D), lambda i: (i, 0)),
                   pl.BlockSpec((_BN, 128), lambda i: (i, 0)),
                   _full((1, 2 * D))],
        out_shape=[jax.ShapeDtypeStruct((N, D), F32),
                   jax.ShapeDtypeStruct((N, 128), F32),
                   jax.ShapeDtypeStruct((1, 2 * D), F32)],
    )(xin, t2, *[wt[n] for n in names])


def _tc_edge_encode(ein, pein, phin, ss_e, wt):
    names = ["ew1", "eb1", "ew2", "eb2", "ew3", "eb3",
             "elg", "elb", "eag", "eab",
             "pew1", "peb1", "pew2", "peb2",
             "phw1", "phb1", "phw2", "phb2"]

    def body(eb, peb, phb, sse, ew1, eb1, ew2, eb2, ew3, eb3,
             elg, elb, eag, eab, pew1, peb1, pew2, peb2,
             phw1, phb1, phw2, phb2, he_o, pe_o, ph_o):
        he = eb[...]
        he = jax.nn.relu(he @ ew1[...] + eb1[...])
        he = jax.nn.relu(he @ ew2[...] + eb2[...])
        he = jax.nn.relu(he @ ew3[...] + eb3[...])
        he = _ln_in(he, elg[...], elb[...])
        ss = sse[...]
        he = _ln_in(he, eag[...], eab[...]) * (1.0 + ss[:, :D]) + ss[:, D:]
        he_o[...] = he
        pe_o[...] = jax.nn.relu(peb[...] @ pew1[...] + peb1[...]) @ pew2[...] + peb2[...]
        ph_o[...] = jax.nn.relu(phb[...] @ phw1[...] + phb1[...]) @ phw2[...] + phb2[...]

    specs = [pl.BlockSpec((_BE, 4), lambda i: (i, 0))] * 3 + [_full((1, 2 * D))]
    specs += [_full(wt[n].shape) for n in names]
    eo = pl.BlockSpec((_BE, D), lambda i: (i, 0))
    return pl.pallas_call(
        body,
        grid=(E // _BE,),
        in_specs=specs,
        out_specs=[eo, eo, eo],
        out_shape=[jax.ShapeDtypeStruct((E, D), F32)] * 3,
    )(ein, pein, phin, ss_e, *[wt[n] for n in names])


def _tc_edge_pass(hsrc, hdst, he, phys, wcat, bcat, elg, elb, expand):
    def body(hs, hd, hb, pb, wc, bc, g, b, ex, we_o, he_o):
        z = jnp.concatenate(
            [hs[...][:, :D].astype(F32), hd[...][:, :D].astype(F32),
             hb[...] + pb[...]], axis=-1)
        u = z @ wc[...] + bc[...]
        m = jax.nn.relu(u[:, 0:D])
        lg = jnp.minimum(u[:, D:D + NH], 60.0)
        eu = jax.nn.relu(u[:, D + NH:])
        he_o[...] = _ln_in(hb[...] + eu, g[...], b[...])
        ev = jnp.exp(lg)
        w = m * (ev @ ex[...])
        we_o[...] = jnp.concatenate(
            [w, ev, jnp.zeros((w.shape[0], 60), F32)], axis=-1)

    eb = pl.BlockSpec((_BE, D), lambda i: (i, 0))
    gb = pl.BlockSpec((_BE, 128), lambda i: (i, 0))
    return pl.pallas_call(
        body,
        grid=(E // _BE,),
        in_specs=[gb, gb, eb, eb, _full(wcat.shape), _full(bcat.shape),
                  _full(elg.shape), _full(elb.shape), _full(expand.shape)],
        out_specs=[gb, eb],
        out_shape=[jax.ShapeDtypeStruct((E, 128), F32),
                   jax.ShapeDtypeStruct((E, D), F32)],
    )(hsrc, hdst, he, phys, wcat, bcat, elg, elb, expand)


def _tc_node_update(h, p0, p1, wu1, wu2, bu, nlg, nlb, expand):
    def body(hb, a0, a1, w1, w2, b, g, bb, ex, h_o, hb_o):
        hv = hb[...]
        s = a0[...][:, 0:D] + a1[...][:, 0:D]
        den = a0[...][:, D:D + NH] + a1[...][:, D:D + NH] + 1e-9
        agg = s / (den @ ex[...])
        u = jax.nn.relu(hv @ w1[...] + agg @ w2[...] + b[...])
        hn = _ln_in(hv + u, g[...], bb[...])
        h_o[...] = hn
        hb_o[...] = jnp.concatenate(
            [hn, jnp.zeros((hn.shape[0], 128 - D), F32)], axis=-1)

    nb = pl.BlockSpec((_BN, D), lambda i: (i, 0))
    pb = pl.BlockSpec((_BN, 128), lambda i: (i, 0))
    return pl.pallas_call(
        body,
        grid=(N // _BN,),
        in_specs=[nb, pb, pb, _full(wu1.shape), _full(wu2.shape),
                  _full(bu.shape), _full(nlg.shape), _full(nlb.shape),
                  _full(expand.shape)],
        out_specs=[nb, pb],
        out_shape=[jax.ShapeDtypeStruct((N, D), F32),
                   jax.ShapeDtypeStruct((N, 128), F32)],
    )(h, p0, p1, wu1, wu2, bu, nlg, nlb, expand)


def _tc_flash_z(qz, z0, h, he, kwn, kbn, vwn, vbn, kwe, kbe, vwe, vbe, wo, bo):
    steps = E // _BE
    nhc = N // _BN

    def body(qz_r, z0_r, h_r, he_b, kwn_r, kbn_r, vwn_r, vbn_r,
             kwe_r, kbe_r, vwe_r, vbe_r, wo_r, bo_r, zn_o,
             m_s, l_s, acc_s):
        i = pl.program_id(0)
        q = qz_r[...]

        @pl.when(i == 0)
        def _init():
            m_s[...] = jnp.full((M, 8), -1e30, F32)
            l_s[...] = jnp.zeros((M, 8), F32)
            acc_s[...] = jnp.zeros((M, D), F32)
            for j in range(nhc):
                hc = h_r[pl.ds(j * _BN, _BN), :]
                k = hc @ kwn_r[...] + kbn_r[...]
                v = hc @ vwn_r[...] + vbn_r[...]
                _flash_upd(q, k, v, m_s, l_s, acc_s)

        k = he_b[...] @ kwe_r[...] + kbe_r[...]
        v = he_b[...] @ vwe_r[...] + vbe_r[...]
        _flash_upd(q, k, v, m_s, l_s, acc_s)

        @pl.when(i == steps - 1)
        def _fin():
            outs = []
            for hh in range(NH):
                sl = slice(hh * HD, (hh + 1) * HD)
                outs.append(acc_s[:, sl] / l_s[:, hh:hh + 1])
            o = jnp.concatenate(outs, axis=-1)
            zn_o[...] = z0_r[...] + o @ wo_r[...] + bo_r[...]

    def _flash_upd(q, k, v, m_s, l_s, acc_s):
        for hh in range(NH):
            sl = slice(hh * HD, (hh + 1) * HD)
            s = lax.dot_general(q[:, sl], k[:, sl],
                                (((1,), (1,)), ((), ()))) * 0.25
            bm = jnp.max(s, axis=-1, keepdims=True)
            m_old = m_s[:, hh:hh + 1]
            m_new = jnp.maximum(m_old, bm)
            alpha = jnp.exp(m_old - m_new)
            p = jnp.exp(s - m_new)
            l_s[:, hh:hh + 1] = l_s[:, hh:hh + 1] * alpha + jnp.sum(
                p, axis=-1, keepdims=True)
            acc_s[:, sl] = acc_s[:, sl] * alpha + lax.dot_general(
                p, v[:, sl], (((1,), (0,)), ((), ())))
            m_s[:, hh:hh + 1] = m_new

    return pl.pallas_call(
        body,
        grid=(steps,),
        in_specs=[_full((M, D)), _full((M, D)), _full((N, D)),
                  pl.BlockSpec((_BE, D), lambda i: (i, 0))]
        + [_full(x.shape) for x in
           (kwn, kbn, vwn, vbn, kwe, kbe, vwe, vbe, wo, bo)],
        out_specs=_full((M, D)),
        out_shape=jax.ShapeDtypeStruct((M, D), F32),
        scratch_shapes=[pltpu.VMEM((M, 8), F32), pltpu.VMEM((M, 8), F32),
                        pltpu.VMEM((M, D), F32)],
    )(qz, z0, h, he, kwn, kbn, vwn, vbn, kwe, kbe, vwe, vbe, wo, bo)


def _tc_mega(xin, zn, bsz, total, wt, dout):
    names = ["pw", "pb", "wq", "bq", "wk", "bk", "wv", "bv", "wo", "bo",
             "lng", "lnb", "f1", "fb1", "f2", "fb2",
             "dlg", "dlb", "d1", "db1", "d2", "db2"]

    def body(xb, zr, pw, pb, wq, bq, wk, bk, wv, bv, wo, bo,
             lng, lnb, f1, fb1, f2, fb2, dlg, dlb, d1, db1, d2, db2, o_r):
        hh0 = xb[...][:, :D] @ pw[...] + pb[...]
        kz = zr[...] @ wk[...] + bk[...]
        vz = zr[...] @ wv[...] + bv[...]
        q = hh0 @ wq[...] + bq[...]
        outs = []
        for hd in range(NH):
            sl = slice(hd * HD, (hd + 1) * HD)
            s = lax.dot_general(q[:, sl], kz[:, sl],
                                (((1,), (1,)), ((), ()))) * 0.25
            a = jax.nn.softmax(s, axis=-1)
            outs.append(lax.dot_general(a, vz[:, sl],
                                        (((1,), (0,)), ((), ()))))
        o = jnp.concatenate(outs, axis=-1)
        hh1 = hh0 + o @ wo[...] + bo[...]
        f = jax.nn.gelu(_ln_in(hh1, lng[...], lnb[...]) @ f1[...] + fb1[...])
        hh2 = hh1 + f @ f2[...] + fb2[...]
        g = jax.nn.gelu(_ln_in(hh2, dlg[...], dlb[...]) @ d1[...] + db1[...])
        o_r[...] = g @ d2[...] + db2[...]

    return pl.pallas_call(
        body,
        grid=(total // bsz,),
        in_specs=[pl.BlockSpec((bsz, xin.shape[1]), lambda i: (i, 0)),
                  _full((M, D))]
        + [_full(wt[n].shape) for n in names],
        out_specs=pl.BlockSpec((bsz, dout), lambda i: (i, 0)),
        out_shape=jax.ShapeDtypeStruct((total, dout), F32),
    )(xin, zn, *[wt[n] for n in names])


def _r1(v):
    return v.reshape(1, -1)


def kernel(x, edge_index, edge_attr, points, batch, t, grad_vec, C_ij, dpos, params):
    p = params
    src = edge_index[0]
    dst = edge_index[1]
    si2 = src.reshape(_CR, _CB)
    di2 = dst.reshape(_CR, _CB)

    expand = jnp.repeat(jnp.eye(NH, dtype=F32), HD, axis=1)

    xin = jnp.concatenate([x, points], axis=-1)
    nwt = {
        "tw1": p["time"][0]["w"], "tb1": _r1(p["time"][0]["b"]),
        "tw2": p["time"][1]["w"], "tb2": _r1(p["time"][1]["b"]),
        "vw1": p["in_v"][0]["w"], "vb1": _r1(p["in_v"][0]["b"]),
        "vw2": p["in_v"][1]["w"], "vb2": _r1(p["in_v"][1]["b"]),
        "vw3": p["in_v"][2]["w"], "vb3": _r1(p["in_v"][2]["b"]),
        "nlg": _r1(p["node_ln"]["g"]), "nlb": _r1(p["node_ln"]["b"]),
        "nag": _r1(p["node_adaln"]["ln"]["g"]), "nab": _r1(p["node_adaln"]["ln"]["b"]),
        "naw": p["node_adaln"]["emb"]["w"], "nabias": _r1(p["node_adaln"]["emb"]["b"]),
        "eaw": p["edge_adaln"]["emb"]["w"], "eabias": _r1(p["edge_adaln"]["emb"]["b"]),
    }
    h, hb, ss_e = _tc_node_encode(xin, t.reshape(1, 1), nwt)

    pein = jnp.concatenate([grad_vec, dpos], axis=-1)
    phin = jnp.concatenate([C_ij, dpos], axis=-1)
    ewt = {
        "ew1": p["in_e"][0]["w"], "eb1": _r1(p["in_e"][0]["b"]),
        "ew2": p["in_e"][1]["w"], "eb2": _r1(p["in_e"][1]["b"]),
        "ew3": p["in_e"][2]["w"], "eb3": _r1(p["in_e"][2]["b"]),
        "elg": _r1(p["edge_ln"]["g"]), "elb": _r1(p["edge_ln"]["b"]),
        "eag": _r1(p["edge_adaln"]["ln"]["g"]), "eab": _r1(p["edge_adaln"]["ln"]["b"]),
        "pew1": p["phys_E"][0]["w"], "peb1": _r1(p["phys_E"][0]["b"]),
        "pew2": p["phys_E"][1]["w"], "peb2": _r1(p["phys_E"][1]["b"]),
        "phw1": p["phys_H"][0]["w"], "phb1": _r1(p["phys_H"][0]["b"]),
        "phw2": p["phys_H"][1]["w"], "phb2": _r1(p["phys_H"][1]["b"]),
    }
    he, physE, physH = _tc_edge_encode(edge_attr, pein, phin, ss_e, ewt)

    for i, lp in enumerate(p["gnn"]):
        phys = physE if i % 2 == 0 else physH
        wcat = jnp.concatenate([lp["Wmsg"]["w"], lp["Watt"]["w"], lp["Wedg"]["w"]],
                               axis=1)
        bcat = _r1(jnp.concatenate([lp["Wmsg"]["b"], lp["Watt"]["b"],
                                    lp["Wedg"]["b"]]))
        hsrc, hdst = _sc_gather2(hb, si2, di2)
        we, he = _tc_edge_pass(hsrc, hdst, he, phys, wcat, bcat,
                               _r1(lp["eln"]["g"]), _r1(lp["eln"]["b"]), expand)
        part = _sc_scatter_add(we, di2)
        h, hb = _tc_node_update(h, part[0], part[1],
                                lp["Wupd"]["w"][:D], lp["Wupd"]["w"][D:],
                                _r1(lp["Wupd"]["b"]),
                                _r1(lp["nln"]["g"]), _r1(lp["nln"]["b"]), expand)

    mz = p["mha_z"]
    qz = p["Z"] @ mz["Wq"]["w"] + mz["Wq"]["b"]
    kwn = p["toE"]["w"] @ mz["Wk"]["w"]
    kbn = _r1(p["toE"]["b"] @ mz["Wk"]["w"] + mz["Wk"]["b"])
    vwn = p["toE"]["w"] @ mz["Wv"]["w"]
    vbn = _r1(p["toE"]["b"] @ mz["Wv"]["w"] + mz["Wv"]["b"])
    kwe = p["toH"]["w"] @ mz["Wk"]["w"]
    kbe = _r1(p["toH"]["b"] @ mz["Wk"]["w"] + mz["Wk"]["b"])
    vwe = p["toH"]["w"] @ mz["Wv"]["w"]
    vbe = _r1(p["toH"]["b"] @ mz["Wv"]["w"] + mz["Wv"]["b"])
    zn = _tc_flash_z(qz, p["Z"], h, he, kwn, kbn, vwn, vbn, kwe, kbe, vwe, vbe,
                     mz["Wo"]["w"], _r1(mz["Wo"]["b"]))

    def mega_wt(proj, mha, ffp, lnp_, dec):
        return {
            "pw": proj["w"], "pb": _r1(proj["b"]),
            "wq": mha["Wq"]["w"], "bq": _r1(mha["Wq"]["b"]),
            "wk": mha["Wk"]["w"], "bk": _r1(mha["Wk"]["b"]),
            "wv": mha["Wv"]["w"], "bv": _r1(mha["Wv"]["b"]),
            "wo": mha["Wo"]["w"], "bo": _r1(mha["Wo"]["b"]),
            "lng": _r1(lnp_["g"]), "lnb": _r1(lnp_["b"]),
            "f1": ffp[0]["w"], "fb1": _r1(ffp[0]["b"]),
            "f2": ffp[1]["w"], "fb2": _r1(ffp[1]["b"]),
            "dlg": _r1(dec["ln"]["g"]), "dlb": _r1(dec["ln"]["b"]),
            "d1": dec["l1"]["w"], "db1": _r1(dec["l1"]["b"]),
            "d2": dec["l2"]["w"], "db2": _r1(dec["l2"]["b"]),
        }

    node_out = _tc_mega(h, zn, _BN, N,
                        mega_wt(p["toE"], p["mha_v"], p["ff_v"], p["lnv"],
                                p["dec_n"]), 1)
    edge_out = _tc_mega(he, zn, _BE, E,
                        mega_wt(p["toH"], p["mha_h"], p["ff_h"], p["lnh"],
                                p["dec_e"]), 2)
    return (node_out, edge_out)


# speedup vs baseline: 1.1640x; 1.1640x over previous
import functools

import jax
import jax.numpy as jnp
import numpy as np
from jax import lax
from jax.experimental import pallas as pl
from jax.experimental.pallas import tpu as pltpu
from jax.experimental.pallas import tpu_sc as plsc

N = 10000
E = 160000
D = 64
T = 64
NH = 4
HD = 16
FF = 256
M = 100
NL = 6

F32 = jnp.float32
BF16 = jnp.bfloat16

_NC = 2
_NS = 16
_NW = _NC * _NS
_CB = 128
_CR = E // _CB
_BASE = _CR // _NW
_EXTRA = _CR - _BASE * _NW
_NPT = N // _NS

def _sc_mesh():
    return plsc.VectorSubcoreMesh(core_axis_name="c", subcore_axis_name="s",
                                  num_cores=_NC, num_subcores=_NS)


def _ln_in(xb, g, b):
    m = jnp.mean(xb, axis=-1, keepdims=True)
    v = jnp.mean((xb - m) ** 2, axis=-1, keepdims=True)
    return (xb - m) / jnp.sqrt(v + 1e-5) * g + b


def _sc_gather2(hb, si2, di2):
    @functools.partial(
        pl.kernel,
        out_type=[jax.ShapeDtypeStruct((E, 128), F32),
                  jax.ShapeDtypeStruct((E, 128), F32)],
        mesh=_sc_mesh(),
        scratch_types=[
            pltpu.VMEM((_CB,), jnp.int32),
            pltpu.VMEM((_CB,), jnp.int32),
            pltpu.VMEM((_CB,), jnp.int32),
            pltpu.VMEM((_CB,), jnp.int32),
            pltpu.VMEM((_CB, 128), F32),
            pltpu.VMEM((_CB, 128), F32),
            pltpu.VMEM((_CB, 128), F32),
            pltpu.VMEM((_CB, 128), F32),
            pltpu.SemaphoreType.DMA,
            pltpu.SemaphoreType.DMA,
            pltpu.SemaphoreType.DMA,
            pltpu.SemaphoreType.DMA,
        ],
    )
    def gk(h_hbm, si_hbm, di_hbm, os_hbm, od_hbm,
           sib0, dib0, sib1, dib1, sbuf0, dbuf0, sbuf1, dbuf1,
           ssem0, dsem0, ssem1, dsem1):
        cid = lax.axis_index("c")
        sid = lax.axis_index("s")
        wid = sid * _NC + cid
        nk = jnp.where(wid < _EXTRA, _BASE + 1, _BASE)

        slots = ((sib0, dib0, sbuf0, dbuf0, ssem0, dsem0),
                 (sib1, dib1, sbuf1, dbuf1, ssem1, dsem1))

        def launch(k, slot):
            sib, dib, sbuf, dbuf, ssem, dsem = slot
            r = wid + _NW * k
            pltpu.sync_copy(si_hbm.at[r], sib)
            pltpu.sync_copy(di_hbm.at[r], dib)
            pltpu.async_copy(h_hbm.at[sib], sbuf, ssem)
            pltpu.async_copy(h_hbm.at[dib], dbuf, dsem)

        def drain(k, slot):
            sib, dib, sbuf, dbuf, ssem, dsem = slot
            r = wid + _NW * k
            pltpu.make_async_copy(h_hbm.at[sib], sbuf, ssem).wait()
            pltpu.sync_copy(sbuf, os_hbm.at[pl.ds(r * _CB, _CB)])
            pltpu.make_async_copy(h_hbm.at[dib], dbuf, dsem).wait()
            pltpu.sync_copy(dbuf, od_hbm.at[pl.ds(r * _CB, _CB)])

        launch(0, slots[0])

        def body(j, carry):
            for b in range(2):
                k = 2 * j + b
                kn = k + 1

                @pl.when(kn < nk)
                def _l():
                    launch(kn, slots[1 - b])

                @pl.when(k < nk)
                def _d():
                    drain(k, slots[b])

            return carry

        lax.fori_loop(0, (nk + 1) // 2, body, 0)

    return gk(hb, si2, di2)


def _sc_scatter_add(we, di2):
    @functools.partial(
        pl.kernel,
        out_type=jax.ShapeDtypeStruct((_NC, N, 128), F32),
        mesh=_sc_mesh(),
        scratch_types=[
            pltpu.VMEM((_CB,), jnp.int32),
            pltpu.VMEM((_CB,), jnp.int32),
            pltpu.VMEM((_CB, 128), F32),
            pltpu.VMEM((_CB, 128), F32),
            pltpu.SemaphoreType.DMA,
            pltpu.SemaphoreType.DMA,
            pltpu.VMEM_SHARED((N, 128), F32),
        ],
    )
    def sk(we_hbm, di_hbm, out_hbm, dib0, dib1, wbuf0, wbuf1,
           wsem0, wsem1, shared):
        cid = lax.axis_index("c")
        sid = lax.axis_index("s")
        wid = sid * _NC + cid
        nk = jnp.where(wid < _EXTRA, _BASE + 1, _BASE)

        def zb(i, carry):
            r = i // 8
            c2 = (i % 8) * 16
            wbuf0[r, pl.ds(c2, 16)] = jnp.zeros((16,), F32)
            return carry

        lax.fori_loop(0, _CB * 8, zb, 0)
        start = jnp.minimum(624 * sid, N - 640)
        for j in range(5):
            pltpu.sync_copy(wbuf0, shared.at[pl.ds(start + j * _CB, _CB)])
        plsc.subcore_barrier()

        slots = ((dib0, wbuf0, wsem0), (dib1, wbuf1, wsem1))

        def launch(k, slot):
            dib, wbuf, wsem = slot
            r = wid + _NW * k
            pltpu.async_copy(we_hbm.at[pl.ds(r * _CB, _CB)], wbuf, wsem)
            pltpu.sync_copy(di_hbm.at[r], dib)

        def drain(k, slot):
            dib, wbuf, wsem = slot
            r = wid + _NW * k
            pltpu.make_async_copy(we_hbm.at[pl.ds(r * _CB, _CB)], wbuf,
                                  wsem).wait()
            pltpu.sync_copy(wbuf, shared.at[dib], add=True)

        launch(0, slots[0])

        def body(j, carry):
            for b in range(2):
                k = 2 * j + b
                kn = k + 1

                @pl.when(kn < nk)
                def _l():
                    launch(kn, slots[1 - b])

                @pl.when(k < nk)
                def _d():
                    drain(k, slots[b])

            return carry

        lax.fori_loop(0, (nk + 1) // 2, body, 0)
        plsc.subcore_barrier()
        for j in range(5):
            pltpu.sync_copy(shared.at[pl.ds(start + j * _CB, _CB)],
                            out_hbm.at[cid].at[pl.ds(start + j * _CB, _CB)])

    return sk(we, di2)


_BN = 2000
_BE = 4000


def _full(shape):
    return pl.BlockSpec(shape, lambda i: tuple(0 for _ in shape))


def _tc_node_encode(xin, ss_n, wt):
    names = ["vw1", "vb1", "vw2", "vb2", "vw3", "vb3",
             "nlg", "nlb", "nag", "nab"]

    def body(xb, ssn, vw1, vb1, vw2, vb2, vw3, vb3,
             nlg, nlb, nag, nab, h_o, hb_o):
        h = xb[...]
        h = jax.nn.relu(h @ vw1[...] + vb1[...])
        h = jax.nn.relu(h @ vw2[...] + vb2[...])
        h = jax.nn.relu(h @ vw3[...] + vb3[...])
        h = _ln_in(h, nlg[...], nlb[...])
        ss = ssn[...]
        h = _ln_in(h, nag[...], nab[...]) * (1.0 + ss[:, :D]) + ss[:, D:]
        h_o[...] = h
        hb_o[...] = jnp.concatenate(
            [h, jnp.zeros((h.shape[0], 128 - D), F32)], axis=-1)

    specs = [pl.BlockSpec((_BN, 4), lambda i: (i, 0)), _full((1, 2 * D))]
    specs += [_full(wt[n].shape) for n in names]
    return pl.pallas_call(
        body,
        grid=(N // _BN,),
        in_specs=specs,
        out_specs=[pl.BlockSpec((_BN, D), lambda i: (i, 0)),
                   pl.BlockSpec((_BN, 128), lambda i: (i, 0))],
        out_shape=[jax.ShapeDtypeStruct((N, D), F32),
                   jax.ShapeDtypeStruct((N, 128), F32)],
    )(xin, ss_n, *[wt[n] for n in names])


def _tc_edge_encode(ein, pein, phin, ss_e, wt):
    names = ["ew1", "eb1", "ew2", "eb2", "ew3", "eb3",
             "elg", "elb", "eag", "eab",
             "pew1", "peb1", "pew2", "peb2",
             "phw1", "phb1", "phw2", "phb2"]

    def body(eb, peb, phb, sse, ew1, eb1, ew2, eb2, ew3, eb3,
             elg, elb, eag, eab, pew1, peb1, pew2, peb2,
             phw1, phb1, phw2, phb2, he_o, pe_o, ph_o):
        he = eb[...]
        he = jax.nn.relu(he @ ew1[...] + eb1[...])
        he = jax.nn.relu(he @ ew2[...] + eb2[...])
        he = jax.nn.relu(he @ ew3[...] + eb3[...])
        he = _ln_in(he, elg[...], elb[...])
        ss = sse[...]
        he = _ln_in(he, eag[...], eab[...]) * (1.0 + ss[:, :D]) + ss[:, D:]
        he_o[...] = he
        pe_o[...] = jax.nn.relu(peb[...] @ pew1[...] + peb1[...]) @ pew2[...] + peb2[...]
        ph_o[...] = jax.nn.relu(phb[...] @ phw1[...] + phb1[...]) @ phw2[...] + phb2[...]

    specs = [pl.BlockSpec((_BE, 4), lambda i: (i, 0))] * 3 + [_full((1, 2 * D))]
    specs += [_full(wt[n].shape) for n in names]
    eo = pl.BlockSpec((_BE, D), lambda i: (i, 0))
    return pl.pallas_call(
        body,
        grid=(E // _BE,),
        in_specs=specs,
        out_specs=[eo, eo, eo],
        out_shape=[jax.ShapeDtypeStruct((E, D), F32)] * 3,
    )(ein, pein, phin, ss_e, *[wt[n] for n in names])


def _tc_edge_pass(hsrc, hdst, he, phys, wcat, bcat, elg, elb, expand):
    def body(hs, hd, hb, pb, wc, bc, g, b, ex, we_o, he_o):
        z = jnp.concatenate(
            [hs[...][:, :D], hd[...][:, :D], hb[...] + pb[...]], axis=-1)
        u = z @ wc[...] + bc[...]
        m = jax.nn.relu(u[:, 0:D])
        lg = jnp.minimum(u[:, D:D + NH], 60.0)
        eu = jax.nn.relu(u[:, D + NH:])
        he_o[...] = _ln_in(hb[...] + eu, g[...], b[...])
        ev = jnp.exp(lg)
        w = m * (ev @ ex[...])
        we_o[...] = jnp.concatenate(
            [w, ev, jnp.zeros((w.shape[0], 60), F32)], axis=-1)

    eb = pl.BlockSpec((_BE, D), lambda i: (i, 0))
    gb = pl.BlockSpec((_BE, 128), lambda i: (i, 0))
    return pl.pallas_call(
        body,
        grid=(E // _BE,),
        in_specs=[gb, gb, eb, eb, _full(wcat.shape), _full(bcat.shape),
                  _full(elg.shape), _full(elb.shape), _full(expand.shape)],
        out_specs=[gb, eb],
        out_shape=[jax.ShapeDtypeStruct((E, 128), F32),
                   jax.ShapeDtypeStruct((E, D), F32)],
    )(hsrc, hdst, he, phys, wcat, bcat, elg, elb, expand)


def _tc_node_update(h, p0, p1, wu, bu, nlg, nlb, expand):
    def body(hb, a0, a1, w, b, g, bb, ex, h_o, hb_o):
        hv = hb[...]
        s = a0[...][:, 0:D] + a1[...][:, 0:D]
        den = a0[...][:, D:D + NH] + a1[...][:, D:D + NH] + 1e-9
        agg = s / (den @ ex[...])
        u = jax.nn.relu(jnp.concatenate([hv, agg], axis=-1) @ w[...] + b[...])
        hn = _ln_in(hv + u, g[...], bb[...])
        h_o[...] = hn
        hb_o[...] = jnp.concatenate(
            [hn, jnp.zeros((hn.shape[0], 128 - D), F32)], axis=-1)

    nb = pl.BlockSpec((_BN, D), lambda i: (i, 0))
    pb = pl.BlockSpec((_BN, 128), lambda i: (i, 0))
    return pl.pallas_call(
        body,
        grid=(N // _BN,),
        in_specs=[nb, pb, pb, _full(wu.shape),
                  _full(bu.shape), _full(nlg.shape), _full(nlb.shape),
                  _full(expand.shape)],
        out_specs=[nb, pb],
        out_shape=[jax.ShapeDtypeStruct((N, D), F32),
                   jax.ShapeDtypeStruct((N, 128), F32)],
    )(h, p0, p1, wu, bu, nlg, nlb, expand)


def _tc_flash_z(qz, z0, h, he, tew, teb, thw, thb, kw, kb, vw, vb, wo, bo):
    steps = E // _BE
    nhc = N // _BN

    def body(qz_r, z0_r, h_r, he_b, tew_r, teb_r, thw_r, thb_r,
             kw_r, kb_r, vw_r, vb_r, wo_r, bo_r, zn_o,
             l_s, acc_s):
        i = pl.program_id(0)
        q = qz_r[...]

        @pl.when(i == 0)
        def _init():
            l_s[...] = jnp.zeros((M, 8), F32)
            acc_s[...] = jnp.zeros((M, D), F32)
            for j in range(nhc):
                hv = h_r[pl.ds(j * _BN, _BN), :] @ tew_r[...] + teb_r[...]
                k = hv @ kw_r[...] + kb_r[...]
                v = hv @ vw_r[...] + vb_r[...]
                _flash_upd(q, k, v, l_s, acc_s)

        hh = he_b[...] @ thw_r[...] + thb_r[...]
        k = hh @ kw_r[...] + kb_r[...]
        v = hh @ vw_r[...] + vb_r[...]
        _flash_upd(q, k, v, l_s, acc_s)

        @pl.when(i == steps - 1)
        def _fin():
            outs = []
            for hh in range(NH):
                sl = slice(hh * HD, (hh + 1) * HD)
                outs.append(acc_s[:, sl] / l_s[:, hh:hh + 1])
            o = jnp.concatenate(outs, axis=-1)
            zn_o[...] = z0_r[...] + o @ wo_r[...] + bo_r[...]

    def _flash_upd(q, k, v, l_s, acc_s):
        for hh in range(NH):
            sl = slice(hh * HD, (hh + 1) * HD)
            s = lax.dot_general(q[:, sl], k[:, sl],
                                (((1,), (1,)), ((), ()))) * 0.25
            p = jnp.exp(s)
            l_s[:, hh:hh + 1] = l_s[:, hh:hh + 1] + jnp.sum(
                p, axis=-1, keepdims=True)
            acc_s[:, sl] = acc_s[:, sl] + lax.dot_general(
                p, v[:, sl], (((1,), (0,)), ((), ())))

    return pl.pallas_call(
        body,
        grid=(steps,),
        in_specs=[_full((M, D)), _full((M, D)), _full((N, D)),
                  pl.BlockSpec((_BE, D), lambda i: (i, 0))]
        + [_full(x.shape) for x in
           (tew, teb, thw, thb, kw, kb, vw, vb, wo, bo)],
        out_specs=_full((M, D)),
        out_shape=jax.ShapeDtypeStruct((M, D), F32),
        scratch_shapes=[pltpu.VMEM((M, 8), F32),
                        pltpu.VMEM((M, D), F32)],
    )(qz, z0, h, he, tew, teb, thw, thb, kw, kb, vw, vb, wo, bo)


def _tc_mega(xin, zn, bsz, total, wt, dout):
    names = ["pw", "pb", "wq", "bq", "wk", "bk", "wv", "bv", "wo", "bo",
             "lng", "lnb", "f1", "fb1", "f2", "fb2",
             "dlg", "dlb", "d1", "db1", "d2", "db2"]

    def body(xb, zr, pw, pb, wq, bq, wk, bk, wv, bv, wo, bo,
             lng, lnb, f1, fb1, f2, fb2, dlg, dlb, d1, db1, d2, db2, o_r):
        hh0 = xb[...][:, :D] @ pw[...] + pb[...]
        kz = zr[...] @ wk[...] + bk[...]
        vz = zr[...] @ wv[...] + bv[...]
        q = hh0 @ wq[...] + bq[...]
        outs = []
        for hd in range(NH):
            sl = slice(hd * HD, (hd + 1) * HD)
            s = lax.dot_general(q[:, sl], kz[:, sl],
                                (((1,), (1,)), ((), ()))) * 0.25
            a = jax.nn.softmax(s, axis=-1)
            outs.append(lax.dot_general(a, vz[:, sl],
                                        (((1,), (0,)), ((), ()))))
        o = jnp.concatenate(outs, axis=-1)
        hh1 = hh0 + o @ wo[...] + bo[...]
        f = jax.nn.gelu(_ln_in(hh1, lng[...], lnb[...]) @ f1[...] + fb1[...])
        hh2 = hh1 + f @ f2[...] + fb2[...]
        g = jax.nn.gelu(_ln_in(hh2, dlg[...], dlb[...]) @ d1[...] + db1[...])
        o_r[...] = g @ d2[...] + db2[...]

    return pl.pallas_call(
        body,
        grid=(total // bsz,),
        in_specs=[pl.BlockSpec((bsz, xin.shape[1]), lambda i: (i, 0)),
                  _full((M, D))]
        + [_full(wt[n].shape) for n in names],
        out_specs=pl.BlockSpec((bsz, dout), lambda i: (i, 0)),
        out_shape=jax.ShapeDtypeStruct((total, dout), F32),
    )(xin, zn, *[wt[n] for n in names])


def _r1(v):
    return v.reshape(1, -1)


def kernel(x, edge_index, edge_attr, points, batch, t, grad_vec, C_ij, dpos, params):
    p = params
    src = edge_index[0]
    dst = edge_index[1]
    si2 = src.reshape(_CR, _CB)
    di2 = dst.reshape(_CR, _CB)

    expand = jnp.repeat(jnp.eye(NH, dtype=F32), HD, axis=1)

    tau = jax.nn.silu(t.reshape(1, 1) @ p["time"][0]["w"]
                      + p["time"][0]["b"]) @ p["time"][1]["w"] + p["time"][1]["b"]
    ss_n = tau @ p["node_adaln"]["emb"]["w"] + p["node_adaln"]["emb"]["b"]
    ss_e = tau @ p["edge_adaln"]["emb"]["w"] + p["edge_adaln"]["emb"]["b"]

    xin = jnp.concatenate([x, points], axis=-1)
    nwt = {
        "vw1": p["in_v"][0]["w"], "vb1": _r1(p["in_v"][0]["b"]),
        "vw2": p["in_v"][1]["w"], "vb2": _r1(p["in_v"][1]["b"]),
        "vw3": p["in_v"][2]["w"], "vb3": _r1(p["in_v"][2]["b"]),
        "nlg": _r1(p["node_ln"]["g"]), "nlb": _r1(p["node_ln"]["b"]),
        "nag": _r1(p["node_adaln"]["ln"]["g"]), "nab": _r1(p["node_adaln"]["ln"]["b"]),
    }
    h, hb = _tc_node_encode(xin, ss_n, nwt)

    pein = jnp.concatenate([grad_vec, dpos], axis=-1)
    phin = jnp.concatenate([C_ij, dpos], axis=-1)
    ewt = {
        "ew1": p["in_e"][0]["w"], "eb1": _r1(p["in_e"][0]["b"]),
        "ew2": p["in_e"][1]["w"], "eb2": _r1(p["in_e"][1]["b"]),
        "ew3": p["in_e"][2]["w"], "eb3": _r1(p["in_e"][2]["b"]),
        "elg": _r1(p["edge_ln"]["g"]), "elb": _r1(p["edge_ln"]["b"]),
        "eag": _r1(p["edge_adaln"]["ln"]["g"]), "eab": _r1(p["edge_adaln"]["ln"]["b"]),
        "pew1": p["phys_E"][0]["w"], "peb1": _r1(p["phys_E"][0]["b"]),
        "pew2": p["phys_E"][1]["w"], "peb2": _r1(p["phys_E"][1]["b"]),
        "phw1": p["phys_H"][0]["w"], "phb1": _r1(p["phys_H"][0]["b"]),
        "phw2": p["phys_H"][1]["w"], "phb2": _r1(p["phys_H"][1]["b"]),
    }
    he, physE, physH = _tc_edge_encode(edge_attr, pein, phin, ss_e, ewt)

    for i, lp in enumerate(p["gnn"]):
        phys = physE if i % 2 == 0 else physH
        wcat = jnp.concatenate([lp["Wmsg"]["w"], lp["Watt"]["w"], lp["Wedg"]["w"]],
                               axis=1)
        bcat = _r1(jnp.concatenate([lp["Wmsg"]["b"], lp["Watt"]["b"],
                                    lp["Wedg"]["b"]]))
        hsrc, hdst = _sc_gather2(hb, si2, di2)
        we, he = _tc_edge_pass(hsrc, hdst, he, phys, wcat, bcat,
                               _r1(lp["eln"]["g"]), _r1(lp["eln"]["b"]), expand)
        part = _sc_scatter_add(we, di2)
        h, hb = _tc_node_update(h, part[0], part[1],
                                lp["Wupd"]["w"], _r1(lp["Wupd"]["b"]),
                                _r1(lp["nln"]["g"]), _r1(lp["nln"]["b"]), expand)

    mz = p["mha_z"]
    qz = p["Z"] @ mz["Wq"]["w"] + mz["Wq"]["b"]
    zn = _tc_flash_z(qz, p["Z"], h, he,
                     p["toE"]["w"], _r1(p["toE"]["b"]),
                     p["toH"]["w"], _r1(p["toH"]["b"]),
                     mz["Wk"]["w"], _r1(mz["Wk"]["b"]),
                     mz["Wv"]["w"], _r1(mz["Wv"]["b"]),
                     mz["Wo"]["w"], _r1(mz["Wo"]["b"]))

    def mega_wt(proj, mha, ffp, lnp_, dec):
        return {
            "pw": proj["w"], "pb": _r1(proj["b"]),
            "wq": mha["Wq"]["w"], "bq": _r1(mha["Wq"]["b"]),
            "wk": mha["Wk"]["w"], "bk": _r1(mha["Wk"]["b"]),
            "wv": mha["Wv"]["w"], "bv": _r1(mha["Wv"]["b"]),
            "wo": mha["Wo"]["w"], "bo": _r1(mha["Wo"]["b"]),
            "lng": _r1(lnp_["g"]), "lnb": _r1(lnp_["b"]),
            "f1": ffp[0]["w"], "fb1": _r1(ffp[0]["b"]),
            "f2": ffp[1]["w"], "fb2": _r1(ffp[1]["b"]),
            "dlg": _r1(dec["ln"]["g"]), "dlb": _r1(dec["ln"]["b"]),
            "d1": dec["l1"]["w"], "db1": _r1(dec["l1"]["b"]),
            "d2": dec["l2"]["w"], "db2": _r1(dec["l2"]["b"]),
        }

    node_out = _tc_mega(h, zn, _BN, N,
                        mega_wt(p["toE"], p["mha_v"], p["ff_v"], p["lnv"],
                                p["dec_n"]), 1)
    edge_out = _tc_mega(he, zn, _BE, E,
                        mega_wt(p["toH"], p["mha_h"], p["ff_h"], p["lnh"],
                                p["dec_e"]), 2)
    return (node_out, edge_out)
